# baseline (speedup 1.0000x reference)
import functools

import jax
import jax.numpy as jnp
from jax import lax
from jax.experimental import pallas as pl
from jax.experimental.pallas import tpu as pltpu
from jax.experimental.pallas import tpu_sc as plsc

VOCAB = 1000000
D = 64
DP = 128
N = 4096 * 50
NC, NS = 2, 16
NW = NC * NS
N_PER_W = N // NW
CH = 400
STEPS = N_PER_W // CH
NGP = STEPS // 2

_mesh = plsc.VectorSubcoreMesh(core_axis_name="c", subcore_axis_name="s")


@functools.partial(
    pl.kernel,
    mesh=_mesh,
    out_type=jax.ShapeDtypeStruct((N, D), jnp.float32),
    scratch_types=[
        pltpu.VMEM((STEPS, CH), jnp.int32),
        pltpu.VMEM((2, CH, DP), jnp.float32),
        pltpu.SemaphoreType.DMA,
        pltpu.SemaphoreType.DMA,
        pltpu.SemaphoreType.DMA,
        pltpu.SemaphoreType.DMA,
    ],
    compiler_params=pltpu.CompilerParams(use_tc_tiling_on_sc=False),
)
def _emb_lookup(idx_hbm, table_hbm, out_hbm, idx_v, rows_v,
                g0sem, g1sem, s0sem, s1sem):
    wid = lax.axis_index("s") * NC + lax.axis_index("c")
    base = wid * N_PER_W
    pltpu.sync_copy(idx_hbm.at[wid], idx_v)

    def fire(g, h, sem):
        pltpu.async_copy(table_hbm.at[idx_v.at[g]], rows_v.at[h], sem)

    def drain_gather(h, sem):
        pltpu.make_async_copy(
            table_hbm.at[idx_v.at[0]], rows_v.at[h], sem).wait()

    def scatter(g, h, sem):
        pltpu.async_copy(
            rows_v.at[h, :, pl.ds(0, D)],
            out_hbm.at[pl.ds(base + g * CH, CH)], sem)

    def drain_scatter(h, sem):
        pltpu.make_async_copy(
            rows_v.at[h, :, pl.ds(0, D)],
            out_hbm.at[pl.ds(base, CH)], sem).wait()

    fire(0, 0, g0sem)

    def pair(p, _):
        g0 = 2 * p
        g1 = g0 + 1

        @pl.when(p > 0)
        def _():
            drain_scatter(1, s1sem)

        fire(g1, 1, g1sem)
        drain_gather(0, g0sem)
        scatter(g0, 0, s0sem)

        @pl.when(p + 1 < NGP)
        def _():
            drain_scatter(0, s0sem)
            fire(g0 + 2, 0, g0sem)

        drain_gather(1, g1sem)
        scatter(g1, 1, s1sem)
        return _

    lax.fori_loop(0, NGP, pair, None)
    drain_scatter(0, s0sem)
    drain_scatter(1, s1sem)


def kernel(x, lut):
    idx = x.reshape(N).astype(jnp.int32).reshape(NW, STEPS, CH)
    lut_p = jnp.pad(lut, ((0, 0), (0, DP - D)))
    out = _emb_lookup(idx, lut_p)
    return out.reshape(x.shape[0], x.shape[1], D)
